# baseline (speedup 1.0000x reference)
import functools

import jax
import jax.numpy as jnp
from jax import lax
from jax.experimental import pallas as pl
from jax.experimental.pallas import tpu as pltpu
from jax.experimental.pallas import tpu_sc as plsc

D = 128
K = 100000
NC = 2
NS = 16
NW = NC * NS
L = 16
RPW = K // NW
CH = 125
NCHUNK = RPW // CH
NBLK = (CH + L - 1) // L
IMAX = 2147483647


def _sqrt16(s):
    i = plsc.bitcast(s, jnp.int32)
    r = plsc.bitcast(jnp.int32(0x5F3759DF) - (i >> 1), jnp.float32)
    for _ in range(4):
        r = r * (1.5 - 0.5 * s * r * r)
    return s * r


def _sc_body(z_hbm, cent_hbm, sims_hbm, rows_hbm, zv, zs, buf, bsim, brow, sems):
    cid = lax.axis_index("c")
    sid = lax.axis_index("s")
    wid = sid * NC + cid
    base = wid * RPW
    iota = lax.iota(jnp.int32, L)

    pltpu.sync_copy(z_hbm, zv)
    nasq_parts = jnp.zeros((L,), jnp.float32)
    for j in range(D // L):
        zc = zv[pl.ds(j * L, L)]
        nasq_parts = nasq_parts + zc * zc
        for t in range(L):
            zs[j * L + t] = jnp.sum(jnp.where(iota == t, zc, 0.0))
    na = _sqrt16(jnp.full((L,), jnp.sum(nasq_parts), jnp.float32))

    bsim[...] = jnp.full((L,), -jnp.inf, jnp.float32)
    brow[...] = jnp.full((L,), IMAX, jnp.int32)

    pltpu.async_copy(cent_hbm.at[pl.ds(base, CH)], buf.at[0], sems.at[0])

    def chunk_body(c, _):
        p = lax.rem(c, 2)
        pltpu.make_async_copy(
            cent_hbm.at[pl.ds(base + c * CH, CH)], buf.at[p], sems.at[p]
        ).wait()

        @pl.when(c + 1 < NCHUNK)
        def _prefetch():
            pltpu.async_copy(
                cent_hbm.at[pl.ds(base + (c + 1) * CH, CH)],
                buf.at[1 - p],
                sems.at[1 - p],
            )

        pv = jnp.full((L,), p, jnp.int32)

        def blk_body(b, _):
            r0 = b * L
            ri = jnp.minimum(r0 + iota, CH - 1)
            acc_d = jnp.zeros((L,), jnp.float32)
            acc_n = jnp.zeros((L,), jnp.float32)
            for d in range(D):
                v = plsc.load_gather(
                    buf, [pv, ri, jnp.full((L,), d, jnp.int32)]
                )
                acc_d = acc_d + v * zs[d]
                acc_n = acc_n + v * v
            nb = _sqrt16(acc_n)
            denom = jnp.maximum(na * nb, 1e-8)
            sims = acc_d / denom
            sims = jnp.where(r0 + iota < CH, sims, -jnp.inf)
            grow = base + c * CH + r0 + iota
            cur = bsim[...]
            upd = sims > cur
            bsim[...] = jnp.where(upd, sims, cur)
            brow[...] = jnp.where(upd, grow, brow[...])
            return 0

        lax.fori_loop(0, NBLK, blk_body, 0)
        return 0

    lax.fori_loop(0, NCHUNK, chunk_body, 0)

    pltpu.sync_copy(bsim, sims_hbm.at[wid])
    pltpu.sync_copy(brow, rows_hbm.at[wid])


_sc_call = functools.partial(
    pl.kernel,
    out_type=[
        jax.ShapeDtypeStruct((NW, L), jnp.float32),
        jax.ShapeDtypeStruct((NW, L), jnp.int32),
    ],
    mesh=plsc.VectorSubcoreMesh(
        core_axis_name="c", subcore_axis_name="s", num_cores=NC, num_subcores=NS
    ),
    scratch_types=[
        pltpu.VMEM((D,), jnp.float32),
        pltpu.SMEM((D,), jnp.float32),
        pltpu.VMEM((2, CH, D), jnp.float32),
        pltpu.VMEM((L,), jnp.float32),
        pltpu.VMEM((L,), jnp.int32),
        pltpu.SemaphoreType.DMA((2,)),
    ],
    compiler_params=pltpu.CompilerParams(
        use_tc_tiling_on_sc=False, needs_layout_passes=False
    ),
)(_sc_body)


def _merge_body(sims_ref, rows_ref, out_ref):
    s = sims_ref[...]
    r = rows_ref[...]
    m = jnp.max(s)
    out_ref[0, 0] = jnp.min(jnp.where(s == m, r, IMAX))


def _merge(sims, rows):
    return pl.pallas_call(
        _merge_body,
        out_shape=jax.ShapeDtypeStruct((1, 1), jnp.int32),
        out_specs=pl.BlockSpec(memory_space=pltpu.SMEM),
    )(sims, rows)


def kernel(z, centroids):
    sims, rows = _sc_call(z, centroids)
    return _merge(sims, rows)[0, 0]
